# Initial kernel scaffold; baseline (speedup 1.0000x reference)
import jax
import jax.numpy as jnp
from jax.experimental import pallas as pl


def kernel(x, cos, sin, Wq, Wk, Wv, Wo, Wiq, Wiw, Wik):
    raise NotImplementedError("write your pallas kernel here")


# speedup vs baseline: 8.3264x; 8.3264x over previous
import functools

import jax
import jax.numpy as jnp
from jax.experimental import pallas as pl
from jax.experimental.pallas import tpu as pltpu

B, T, DM = 1, 2048, 1024
H, HKV, DH = 16, 4, 64
TOPK = 64
NIH, IHD = 4, 32
EPS = 1.1920929e-07
SCALE = DH ** -0.5
G = H // HKV

BLK = 256
NBLK = T // BLK
NEG = -1e30


def _rope_rms_head(xh, c, s):
    d = DH // 2
    rot = jnp.concatenate([-xh[:, d:], xh[:, :d]], axis=1)
    r = xh * c + rot * s
    return r * jax.lax.rsqrt(jnp.mean(r * r, axis=-1, keepdims=True) + EPS)


def _mono_key(x):
    b = jax.lax.bitcast_convert_type(x, jnp.int32)
    return jnp.where(b >= 0, b, b ^ jnp.int32(0x7FFFFFFF))


def _kv_kernel(x_ref, cos_ref, sin_ref, w_ref, k_ref, v_ref, ki_ref):
    y = jnp.dot(x_ref[...], w_ref[...], preferred_element_type=jnp.float32)
    c = cos_ref[...]
    s = sin_ref[...]
    for h in range(HKV):
        k_ref[:, h * DH:(h + 1) * DH] = _rope_rms_head(
            y[:, h * DH:(h + 1) * DH], c, s)
    v_ref[...] = y[:, HKV * DH:2 * HKV * DH]
    ki_ref[...] = y[:, 2 * HKV * DH:2 * HKV * DH + IHD]


def _attn_kernel(x_ref, cos_ref, sin_ref, wq_ref, wiq_ref, wiw_ref, wo_ref,
                 k_ref, v_ref, ki_ref, out_ref):
    blk = pl.program_id(0)
    x = x_ref[...]
    c = cos_ref[...]
    s = sin_ref[...]

    yq = jnp.dot(x, wq_ref[...], preferred_element_type=jnp.float32)
    qh = [_rope_rms_head(yq[:, h * DH:(h + 1) * DH], c, s) for h in range(H)]

    qi = jnp.dot(x, wiq_ref[...], preferred_element_type=jnp.float32)
    wi = jnp.dot(x, wiw_ref[...], preferred_element_type=jnp.float32)
    ki = ki_ref[...]
    acc = jnp.zeros((BLK, T), jnp.float32)
    for h in range(NIH):
        raw = jax.lax.dot_general(
            qi[:, h * IHD:(h + 1) * IHD], ki,
            (((1,), (1,)), ((), ())), preferred_element_type=jnp.float32)
        acc = acc + jnp.maximum(raw, 0.0) * wi[:, h:h + 1]

    col = jax.lax.broadcasted_iota(jnp.int32, (BLK, T), 1)
    row = jax.lax.broadcasted_iota(jnp.int32, (BLK, T), 0) + blk * BLK
    valid = col <= row
    masked = jnp.where(valid, acc, -jnp.inf)
    key = _mono_key(masked)

    def cnt_ge(v):
        return jnp.sum((key >= v).astype(jnp.int32), axis=1, keepdims=True)

    v64 = jnp.where(cnt_ge(jnp.zeros((BLK, 1), jnp.int32)) >= TOPK,
                    jnp.int32(0), jnp.int32(-2147483648))
    v64 = jnp.broadcast_to(v64, (BLK, 1))
    for bit in range(30, -1, -1):
        cand = v64 | jnp.int32(1 << bit)
        v64 = jnp.where(cnt_ge(cand) >= TOPK, cand, v64)

    gt = key > v64
    eq = key == v64
    need = TOPK - jnp.sum(gt.astype(jnp.int32), axis=1, keepdims=True)

    lo = jnp.zeros((BLK, 1), jnp.int32)
    hi = jnp.full((BLK, 1), T, jnp.int32)
    eq_i = eq.astype(jnp.int32)
    for _ in range(12):
        mid = (lo + hi) // 2
        cnt = jnp.sum(jnp.where(col < mid, eq_i, 0), axis=1, keepdims=True)
        pred = cnt >= need
        hi = jnp.where(pred, mid, hi)
        lo = jnp.where(pred, lo, mid + 1)
    sel = (gt | (eq & (col < hi))) & valid

    sel_st = jnp.concatenate([sel] * G, axis=0)
    oh = [None] * H
    for n in range(HKV):
        kn = k_ref[:, n * DH:(n + 1) * DH]
        vn = v_ref[:, n * DH:(n + 1) * DH]
        q_st = jnp.concatenate([qh[n * G + g] for g in range(G)], axis=0)
        sc = jax.lax.dot_general(
            q_st, kn, (((1,), (1,)), ((), ())),
            preferred_element_type=jnp.float32) * SCALE
        sc = jnp.where(sel_st, sc, NEG)
        m = jnp.max(sc, axis=1, keepdims=True)
        p = jnp.exp(sc - m)
        p = p / jnp.sum(p, axis=1, keepdims=True)
        o_st = jnp.dot(p, vn, preferred_element_type=jnp.float32)
        for g in range(G):
            oh[n * G + g] = o_st[g * BLK:(g + 1) * BLK, :]

    out_heads = jnp.concatenate(oh, axis=1)
    out_ref[...] = jnp.dot(out_heads, wo_ref[...],
                           preferred_element_type=jnp.float32)


@jax.jit
def kernel(x, cos, sin, Wq, Wk, Wv, Wo, Wiq, Wiw, Wik):
    x2 = x.reshape(T, DM)
    cos2 = cos.reshape(T, DH)
    sin2 = sin.reshape(T, DH)
    wkvi = jnp.concatenate([Wk, Wv, Wik], axis=1)

    full = lambda shape: pl.BlockSpec(shape, lambda i: (0, 0))
    rows = lambda w: pl.BlockSpec((BLK, w), lambda i: (i, 0))

    k, v, ki = pl.pallas_call(
        _kv_kernel,
        grid=(NBLK,),
        in_specs=[rows(DM), rows(DH), rows(DH), full((DM, 2 * HKV * DH + IHD))],
        out_specs=[rows(HKV * DH), rows(HKV * DH), rows(IHD)],
        out_shape=[
            jax.ShapeDtypeStruct((T, HKV * DH), jnp.float32),
            jax.ShapeDtypeStruct((T, HKV * DH), jnp.float32),
            jax.ShapeDtypeStruct((T, IHD), jnp.float32),
        ],
        compiler_params=pltpu.CompilerParams(
            dimension_semantics=("arbitrary",)),
    )(x2, cos2, sin2, wkvi)

    out = pl.pallas_call(
        _attn_kernel,
        grid=(NBLK,),
        in_specs=[
            rows(DM), rows(DH), rows(DH),
            full((DM, H * DH)), full((DM, NIH * IHD)), full((DM, NIH)),
            full((H * DH, DM)),
            full((T, HKV * DH)), full((T, HKV * DH)), full((T, IHD)),
        ],
        out_specs=rows(DM),
        out_shape=jax.ShapeDtypeStruct((T, DM), jnp.float32),
        compiler_params=pltpu.CompilerParams(
            dimension_semantics=("arbitrary",)),
    )(x2, cos2, sin2, Wq, Wiq, Wiw, Wo, k, v, ki)

    return out.reshape(B, T, DM)


# speedup vs baseline: 8.3908x; 1.0077x over previous
import functools

import jax
import jax.numpy as jnp
from jax.experimental import pallas as pl
from jax.experimental.pallas import tpu as pltpu

B, T, DM = 1, 2048, 1024
H, HKV, DH = 16, 4, 64
TOPK = 64
NIH, IHD = 4, 32
EPS = 1.1920929e-07
SCALE = DH ** -0.5
G = H // HKV

BLK = 256
NBLK = T // BLK
NEG = -1e30


def _rope_rms_head(xh, c, s):
    d = DH // 2
    rot = jnp.concatenate([-xh[:, d:], xh[:, :d]], axis=1)
    r = xh * c + rot * s
    return r * jax.lax.rsqrt(jnp.mean(r * r, axis=-1, keepdims=True) + EPS)


def _mono_key(x):
    b = jax.lax.bitcast_convert_type(x, jnp.int32)
    return jnp.where(b >= 0, b, b ^ jnp.int32(0x7FFFFFFF))


def _kv_kernel(x_ref, cos_ref, sin_ref, w_ref, wik_ref, k_ref, v_ref, ki_ref):
    x = x_ref[...]
    y = jnp.dot(x.astype(jnp.bfloat16), w_ref[...].astype(jnp.bfloat16),
                preferred_element_type=jnp.float32)
    c = cos_ref[...]
    s = sin_ref[...]
    for h in range(HKV):
        k_ref[:, h * DH:(h + 1) * DH] = _rope_rms_head(
            y[:, h * DH:(h + 1) * DH], c, s)
    v_ref[...] = y[:, HKV * DH:2 * HKV * DH]
    ki_ref[...] = jnp.dot(x, wik_ref[...], preferred_element_type=jnp.float32)


def _attn_kernel(x_ref, cos_ref, sin_ref, wq_ref, wiq_ref, wiw_ref, wo_ref,
                 k_ref, v_ref, ki_ref, out_ref):
    blk = pl.program_id(0)
    x = x_ref[...]
    c = cos_ref[...]
    s = sin_ref[...]

    yq = jnp.dot(x.astype(jnp.bfloat16), wq_ref[...].astype(jnp.bfloat16),
                 preferred_element_type=jnp.float32)
    qh = [_rope_rms_head(yq[:, h * DH:(h + 1) * DH], c, s) for h in range(H)]

    qi = jnp.dot(x, wiq_ref[...], preferred_element_type=jnp.float32)
    wi = jnp.dot(x, wiw_ref[...], preferred_element_type=jnp.float32)
    ki = ki_ref[...]
    acc = jnp.zeros((BLK, T), jnp.float32)
    for h in range(NIH):
        raw = jax.lax.dot_general(
            qi[:, h * IHD:(h + 1) * IHD], ki,
            (((1,), (1,)), ((), ())), preferred_element_type=jnp.float32)
        acc = acc + jnp.maximum(raw, 0.0) * wi[:, h:h + 1]

    col = jax.lax.broadcasted_iota(jnp.int32, (BLK, T), 1)
    row = jax.lax.broadcasted_iota(jnp.int32, (BLK, T), 0) + blk * BLK
    valid = col <= row
    masked = jnp.where(valid, acc, -jnp.inf)
    key = _mono_key(masked)

    def cnt_ge(v):
        return jnp.sum((key >= v).astype(jnp.int32), axis=1, keepdims=True)

    v64 = jnp.where(cnt_ge(jnp.zeros((BLK, 1), jnp.int32)) >= TOPK,
                    jnp.int32(0), jnp.int32(-2147483648))
    v64 = jnp.broadcast_to(v64, (BLK, 1))
    for bit in range(30, -1, -1):
        cand = v64 | jnp.int32(1 << bit)
        v64 = jnp.where(cnt_ge(cand) >= TOPK, cand, v64)

    gt = key > v64
    eq = key == v64
    need = TOPK - jnp.sum(gt.astype(jnp.int32), axis=1, keepdims=True)

    lo = jnp.zeros((BLK, 1), jnp.int32)
    hi = jnp.full((BLK, 1), T, jnp.int32)
    eq_i = eq.astype(jnp.int32)
    for _ in range(12):
        mid = (lo + hi) // 2
        cnt = jnp.sum(jnp.where(col < mid, eq_i, 0), axis=1, keepdims=True)
        pred = cnt >= need
        hi = jnp.where(pred, mid, hi)
        lo = jnp.where(pred, lo, mid + 1)
    sel = (gt | (eq & (col < hi))) & valid

    sel_st = jnp.concatenate([sel] * G, axis=0)
    oh = [None] * H
    for n in range(HKV):
        kn = k_ref[:, n * DH:(n + 1) * DH]
        vn = v_ref[:, n * DH:(n + 1) * DH]
        q_st = jnp.concatenate([qh[n * G + g] for g in range(G)], axis=0)
        sc = jax.lax.dot_general(
            q_st.astype(jnp.bfloat16), kn.astype(jnp.bfloat16),
            (((1,), (1,)), ((), ())),
            preferred_element_type=jnp.float32) * SCALE
        sc = jnp.where(sel_st, sc, NEG)
        m = jnp.max(sc, axis=1, keepdims=True)
        p = jnp.exp(sc - m)
        p = p / jnp.sum(p, axis=1, keepdims=True)
        o_st = jnp.dot(p.astype(jnp.bfloat16), vn.astype(jnp.bfloat16),
                       preferred_element_type=jnp.float32)
        for g in range(G):
            oh[n * G + g] = o_st[g * BLK:(g + 1) * BLK, :]

    out_heads = jnp.concatenate(oh, axis=1)
    out_ref[...] = jnp.dot(out_heads.astype(jnp.bfloat16),
                           wo_ref[...].astype(jnp.bfloat16),
                           preferred_element_type=jnp.float32)


@jax.jit
def kernel(x, cos, sin, Wq, Wk, Wv, Wo, Wiq, Wiw, Wik):
    x2 = x.reshape(T, DM)
    cos2 = cos.reshape(T, DH)
    sin2 = sin.reshape(T, DH)
    wkv = jnp.concatenate([Wk, Wv], axis=1)

    full = lambda shape: pl.BlockSpec(shape, lambda i: (0, 0))
    rows = lambda w: pl.BlockSpec((BLK, w), lambda i: (i, 0))

    k, v, ki = pl.pallas_call(
        _kv_kernel,
        grid=(NBLK,),
        in_specs=[rows(DM), rows(DH), rows(DH), full((DM, 2 * HKV * DH)),
                  full((DM, IHD))],
        out_specs=[rows(HKV * DH), rows(HKV * DH), rows(IHD)],
        out_shape=[
            jax.ShapeDtypeStruct((T, HKV * DH), jnp.float32),
            jax.ShapeDtypeStruct((T, HKV * DH), jnp.float32),
            jax.ShapeDtypeStruct((T, IHD), jnp.float32),
        ],
        compiler_params=pltpu.CompilerParams(
            dimension_semantics=("arbitrary",)),
    )(x2, cos2, sin2, wkv, Wik)

    out = pl.pallas_call(
        _attn_kernel,
        grid=(NBLK,),
        in_specs=[
            rows(DM), rows(DH), rows(DH),
            full((DM, H * DH)), full((DM, NIH * IHD)), full((DM, NIH)),
            full((H * DH, DM)),
            full((T, HKV * DH)), full((T, HKV * DH)), full((T, IHD)),
        ],
        out_specs=rows(DM),
        out_shape=jax.ShapeDtypeStruct((T, DM), jnp.float32),
        compiler_params=pltpu.CompilerParams(
            dimension_semantics=("arbitrary",)),
    )(x2, cos2, sin2, Wq, Wiq, Wiw, Wo, k, v, ki)

    return out.reshape(B, T, DM)


# speedup vs baseline: 9.1483x; 1.0903x over previous
import functools

import jax
import jax.numpy as jnp
from jax.experimental import pallas as pl
from jax.experimental.pallas import tpu as pltpu

B, T, DM = 1, 2048, 1024
H, HKV, DH = 16, 4, 64
TOPK = 64
NIH, IHD = 4, 32
EPS = 1.1920929e-07
SCALE = DH ** -0.5
G = H // HKV

BLK = 256
NBLK = T // BLK
NEG = -1e30


def _rope_rms_head(xh, c, s):
    d = DH // 2
    rot = jnp.concatenate([-xh[:, d:], xh[:, :d]], axis=1)
    r = xh * c + rot * s
    return r * jax.lax.rsqrt(jnp.mean(r * r, axis=-1, keepdims=True) + EPS)


def _mono_key(x):
    b = jax.lax.bitcast_convert_type(x, jnp.int32)
    return jnp.where(b >= 0, b, b ^ jnp.int32(0x7FFFFFFF))


def _kv_kernel(x_ref, cos_ref, sin_ref, w_ref, wik_ref, k_ref, v_ref, ki_ref):
    x = x_ref[...]
    y = jnp.dot(x.astype(jnp.bfloat16), w_ref[...].astype(jnp.bfloat16),
                preferred_element_type=jnp.float32)
    c = cos_ref[...]
    s = sin_ref[...]
    for h in range(HKV):
        k_ref[:, h * DH:(h + 1) * DH] = _rope_rms_head(
            y[:, h * DH:(h + 1) * DH], c, s)
    v_ref[...] = y[:, HKV * DH:2 * HKV * DH]
    ki_ref[...] = jnp.dot(x, wik_ref[...], preferred_element_type=jnp.float32)


def _attn_kernel(x_ref, cos_ref, sin_ref, wq_ref, wiq_ref, wiw_ref, wo_ref,
                 k_ref, v_ref, ki_ref, out_ref):
    blk = pl.program_id(0)
    x = x_ref[...]
    c = cos_ref[...]
    s = sin_ref[...]

    yq = jnp.dot(x.astype(jnp.bfloat16), wq_ref[...].astype(jnp.bfloat16),
                 preferred_element_type=jnp.float32)
    qh = [_rope_rms_head(yq[:, h * DH:(h + 1) * DH], c, s) for h in range(H)]

    qi = jnp.dot(x, wiq_ref[...], preferred_element_type=jnp.float32)
    wi = jnp.dot(x, wiw_ref[...], preferred_element_type=jnp.float32)
    ki = ki_ref[...]
    acc = jnp.zeros((BLK, T), jnp.float32)
    for h in range(NIH):
        raw = jax.lax.dot_general(
            qi[:, h * IHD:(h + 1) * IHD], ki,
            (((1,), (1,)), ((), ())), preferred_element_type=jnp.float32)
        acc = acc + jnp.maximum(raw, 0.0) * wi[:, h:h + 1]

    col = jax.lax.broadcasted_iota(jnp.int32, (BLK, T), 1)
    row = jax.lax.broadcasted_iota(jnp.int32, (BLK, T), 0) + blk * BLK
    valid = col <= row
    masked = jnp.where(valid, acc, -jnp.inf)
    key = _mono_key(masked)

    ones_col = jnp.ones((T, 1), jnp.float32)

    def cnt_ge(v):
        m = jnp.where(key >= v, 1.0, 0.0)
        return jnp.dot(m, ones_col, preferred_element_type=jnp.float32)

    v64 = jnp.where(cnt_ge(jnp.zeros((BLK, 1), jnp.int32)) >= TOPK,
                    jnp.int32(0), jnp.int32(-2147483648))
    v64 = jnp.broadcast_to(v64, (BLK, 1))
    for bit in range(30, -1, -1):
        cand = v64 | jnp.int32(1 << bit)
        v64 = jnp.where(cnt_ge(cand) >= TOPK, cand, v64)

    gt = key > v64
    eq = key == v64
    gt_f = jnp.where(gt, 1.0, 0.0)
    need = TOPK - jnp.dot(gt_f, ones_col, preferred_element_type=jnp.float32)

    lo = jnp.zeros((BLK, 1), jnp.int32)
    hi = jnp.full((BLK, 1), T, jnp.int32)
    for _ in range(12):
        mid = (lo + hi) // 2
        m = jnp.where(eq & (col < mid), 1.0, 0.0)
        cnt = jnp.dot(m, ones_col, preferred_element_type=jnp.float32)
        pred = cnt >= need
        hi = jnp.where(pred, mid, hi)
        lo = jnp.where(pred, lo, mid + 1)
    sel = (gt | (eq & (col < hi))) & valid

    sel_st = jnp.concatenate([sel] * G, axis=0)
    oh = [None] * H
    for n in range(HKV):
        kn = k_ref[:, n * DH:(n + 1) * DH]
        vn = v_ref[:, n * DH:(n + 1) * DH]
        ve = jnp.concatenate([vn, jnp.ones((T, 1), jnp.float32)], axis=1)
        q_st = jnp.concatenate(
            [qh[n * G + g] for g in range(G)], axis=0) * SCALE
        sc = jax.lax.dot_general(
            q_st.astype(jnp.bfloat16), kn.astype(jnp.bfloat16),
            (((1,), (1,)), ((), ())),
            preferred_element_type=jnp.float32)
        p = jnp.where(sel_st, jnp.exp(sc), 0.0)
        o_st = jnp.dot(p.astype(jnp.bfloat16), ve.astype(jnp.bfloat16),
                       preferred_element_type=jnp.float32)
        o_st = o_st[:, :DH] / o_st[:, DH:DH + 1]
        for g in range(G):
            oh[n * G + g] = o_st[g * BLK:(g + 1) * BLK, :]

    out_heads = jnp.concatenate(oh, axis=1)
    out_ref[...] = jnp.dot(out_heads.astype(jnp.bfloat16),
                           wo_ref[...].astype(jnp.bfloat16),
                           preferred_element_type=jnp.float32)


@jax.jit
def kernel(x, cos, sin, Wq, Wk, Wv, Wo, Wiq, Wiw, Wik):
    x2 = x.reshape(T, DM)
    cos2 = cos.reshape(T, DH)
    sin2 = sin.reshape(T, DH)
    wkv = jnp.concatenate([Wk, Wv], axis=1)

    full = lambda shape: pl.BlockSpec(shape, lambda i: (0, 0))
    rows = lambda w: pl.BlockSpec((BLK, w), lambda i: (i, 0))

    k, v, ki = pl.pallas_call(
        _kv_kernel,
        grid=(NBLK,),
        in_specs=[rows(DM), rows(DH), rows(DH), full((DM, 2 * HKV * DH)),
                  full((DM, IHD))],
        out_specs=[rows(HKV * DH), rows(HKV * DH), rows(IHD)],
        out_shape=[
            jax.ShapeDtypeStruct((T, HKV * DH), jnp.float32),
            jax.ShapeDtypeStruct((T, HKV * DH), jnp.float32),
            jax.ShapeDtypeStruct((T, IHD), jnp.float32),
        ],
        compiler_params=pltpu.CompilerParams(
            dimension_semantics=("arbitrary",)),
    )(x2, cos2, sin2, wkv, Wik)

    out = pl.pallas_call(
        _attn_kernel,
        grid=(NBLK,),
        in_specs=[
            rows(DM), rows(DH), rows(DH),
            full((DM, H * DH)), full((DM, NIH * IHD)), full((DM, NIH)),
            full((H * DH, DM)),
            full((T, HKV * DH)), full((T, HKV * DH)), full((T, IHD)),
        ],
        out_specs=rows(DM),
        out_shape=jax.ShapeDtypeStruct((T, DM), jnp.float32),
        compiler_params=pltpu.CompilerParams(
            dimension_semantics=("arbitrary",)),
    )(x2, cos2, sin2, Wq, Wiq, Wiw, Wo, k, v, ki)

    return out.reshape(B, T, DM)


# speedup vs baseline: 9.3621x; 1.0234x over previous
import functools

import jax
import jax.numpy as jnp
from jax.experimental import pallas as pl
from jax.experimental.pallas import tpu as pltpu

B, T, DM = 1, 2048, 1024
H, HKV, DH = 16, 4, 64
TOPK = 64
NIH, IHD = 4, 32
EPS = 1.1920929e-07
SCALE = DH ** -0.5
G = H // HKV

BLK = 256
NBLK = T // BLK
NEG = -1e30


def _rope_rms_head(xh, c, s):
    d = DH // 2
    rot = jnp.concatenate([-xh[:, d:], xh[:, :d]], axis=1)
    r = xh * c + rot * s
    return r * jax.lax.rsqrt(jnp.mean(r * r, axis=-1, keepdims=True) + EPS)


def _mono_key(x):
    b = jax.lax.bitcast_convert_type(x, jnp.int32)
    return jnp.where(b >= 0, b, b ^ jnp.int32(0x7FFFFFFF))


def _kv_kernel(x_ref, cos_ref, sin_ref, w_ref, wik_ref, k_ref, v_ref, ki_ref):
    x = x_ref[...]
    y = jnp.dot(x.astype(jnp.bfloat16), w_ref[...].astype(jnp.bfloat16),
                preferred_element_type=jnp.float32)
    c = cos_ref[...]
    s = sin_ref[...]
    for h in range(HKV):
        k_ref[:, h * DH:(h + 1) * DH] = _rope_rms_head(
            y[:, h * DH:(h + 1) * DH], c, s)
    v_ref[...] = y[:, HKV * DH:2 * HKV * DH]
    ki_ref[...] = jnp.dot(x, wik_ref[...], preferred_element_type=jnp.float32)


def _make_attn_kernel(width, row0):
    def body(x_ref, cos_ref, sin_ref, wq_ref, wiq_ref, wiw_ref, wo_ref,
             k_ref, v_ref, ki_ref, out_ref):
        blk = pl.program_id(0)
        x = x_ref[...]
        c = cos_ref[...]
        s = sin_ref[...]

        yq = jnp.dot(x.astype(jnp.bfloat16), wq_ref[...].astype(jnp.bfloat16),
                     preferred_element_type=jnp.float32)
        qh = [_rope_rms_head(yq[:, h * DH:(h + 1) * DH], c, s)
              for h in range(H)]

        qi = jnp.dot(x, wiq_ref[...], preferred_element_type=jnp.float32)
        wi = jnp.dot(x, wiw_ref[...], preferred_element_type=jnp.float32)
        ki = ki_ref[...]
        acc = jnp.zeros((BLK, width), jnp.float32)
        for h in range(NIH):
            raw = jax.lax.dot_general(
                qi[:, h * IHD:(h + 1) * IHD], ki,
                (((1,), (1,)), ((), ())), preferred_element_type=jnp.float32)
            acc = acc + jnp.maximum(raw, 0.0) * wi[:, h:h + 1]

        col = jax.lax.broadcasted_iota(jnp.int32, (BLK, width), 1)
        row = jax.lax.broadcasted_iota(jnp.int32, (BLK, width), 0) \
            + (row0 + blk * BLK)
        valid = col <= row
        masked = jnp.where(valid, acc, -jnp.inf)
        key = _mono_key(masked)

        ones_col = jnp.ones((width, 1), jnp.float32)

        def cnt_ge(v):
            m = jnp.where(key >= v, 1.0, 0.0)
            return jnp.dot(m, ones_col, preferred_element_type=jnp.float32)

        v64 = jnp.where(cnt_ge(jnp.zeros((BLK, 1), jnp.int32)) >= TOPK,
                        jnp.int32(0), jnp.int32(-2147483648))
        v64 = jnp.broadcast_to(v64, (BLK, 1))
        for bit in range(30, -1, -1):
            cand = v64 | jnp.int32(1 << bit)
            v64 = jnp.where(cnt_ge(cand) >= TOPK, cand, v64)

        gt = key > v64
        eq = key == v64
        gt_f = jnp.where(gt, 1.0, 0.0)
        need = TOPK - jnp.dot(gt_f, ones_col,
                              preferred_element_type=jnp.float32)

        lo = jnp.zeros((BLK, 1), jnp.int32)
        hi = jnp.full((BLK, 1), width, jnp.int32)
        for _ in range((width + 1).bit_length()):
            mid = (lo + hi) // 2
            m = jnp.where(eq & (col < mid), 1.0, 0.0)
            cnt = jnp.dot(m, ones_col, preferred_element_type=jnp.float32)
            pred = cnt >= need
            hi = jnp.where(pred, mid, hi)
            lo = jnp.where(pred, lo, mid + 1)
        sel = (gt | (eq & (col < hi))) & valid

        sel_st = jnp.concatenate([sel] * G, axis=0)
        oh = [None] * H
        for n in range(HKV):
            kn = k_ref[:, n * DH:(n + 1) * DH]
            vn = v_ref[:, n * DH:(n + 1) * DH]
            ve = jnp.concatenate(
                [vn, jnp.ones((width, 1), jnp.float32)], axis=1)
            q_st = jnp.concatenate(
                [qh[n * G + g] for g in range(G)], axis=0) * SCALE
            sc = jax.lax.dot_general(
                q_st.astype(jnp.bfloat16), kn.astype(jnp.bfloat16),
                (((1,), (1,)), ((), ())),
                preferred_element_type=jnp.float32)
            p = jnp.where(sel_st, jnp.exp(sc), 0.0)
            o_st = jnp.dot(p.astype(jnp.bfloat16), ve.astype(jnp.bfloat16),
                           preferred_element_type=jnp.float32)
            o_st = o_st[:, :DH] / o_st[:, DH:DH + 1]
            for g in range(G):
                oh[n * G + g] = o_st[g * BLK:(g + 1) * BLK, :]

        out_heads = jnp.concatenate(oh, axis=1)
        out_ref[...] = jnp.dot(out_heads.astype(jnp.bfloat16),
                               wo_ref[...].astype(jnp.bfloat16),
                               preferred_element_type=jnp.float32)

    return body


@jax.jit
def kernel(x, cos, sin, Wq, Wk, Wv, Wo, Wiq, Wiw, Wik):
    x2 = x.reshape(T, DM)
    cos2 = cos.reshape(T, DH)
    sin2 = sin.reshape(T, DH)
    wkv = jnp.concatenate([Wk, Wv], axis=1)

    full = lambda shape: pl.BlockSpec(shape, lambda i: (0, 0))
    rows = lambda w: pl.BlockSpec((BLK, w), lambda i: (i, 0))

    k, v, ki = pl.pallas_call(
        _kv_kernel,
        grid=(NBLK,),
        in_specs=[rows(DM), rows(DH), rows(DH), full((DM, 2 * HKV * DH)),
                  full((DM, IHD))],
        out_specs=[rows(HKV * DH), rows(HKV * DH), rows(IHD)],
        out_shape=[
            jax.ShapeDtypeStruct((T, HKV * DH), jnp.float32),
            jax.ShapeDtypeStruct((T, HKV * DH), jnp.float32),
            jax.ShapeDtypeStruct((T, IHD), jnp.float32),
        ],
        compiler_params=pltpu.CompilerParams(
            dimension_semantics=("arbitrary",)),
    )(x2, cos2, sin2, wkv, Wik)

    outs = []
    for p in range(NBLK // 2):
        row0 = p * 2 * BLK
        width = (p + 1) * 2 * BLK
        o = pl.pallas_call(
            _make_attn_kernel(width, row0),
            grid=(2,),
            in_specs=[
                rows(DM), rows(DH), rows(DH),
                full((DM, H * DH)), full((DM, NIH * IHD)), full((DM, NIH)),
                full((H * DH, DM)),
                full((width, HKV * DH)), full((width, HKV * DH)),
                full((width, IHD)),
            ],
            out_specs=rows(DM),
            out_shape=jax.ShapeDtypeStruct((2 * BLK, DM), jnp.float32),
            compiler_params=pltpu.CompilerParams(
                dimension_semantics=("arbitrary",)),
        )(x2[row0:row0 + 2 * BLK], cos2[row0:row0 + 2 * BLK],
          sin2[row0:row0 + 2 * BLK], Wq, Wiq, Wiw, Wo,
          k[:width], v[:width], ki[:width])
        outs.append(o)

    return jnp.concatenate(outs, axis=0).reshape(B, T, DM)


# speedup vs baseline: 12.2327x; 1.3066x over previous
import functools

import jax
import jax.numpy as jnp
from jax.experimental import pallas as pl
from jax.experimental.pallas import tpu as pltpu

B, T, DM = 1, 2048, 1024
H, HKV, DH = 16, 4, 64
TOPK = 64
NIH, IHD = 4, 32
EPS = 1.1920929e-07
SCALE = DH ** -0.5
G = H // HKV

BLK = 256
NBLK = T // BLK
NEG = -1e30


def _rope_rms_head(xh, c, s):
    d = DH // 2
    rot = jnp.concatenate([-xh[:, d:], xh[:, :d]], axis=1)
    r = xh * c + rot * s
    return r * jax.lax.rsqrt(jnp.mean(r * r, axis=-1, keepdims=True) + EPS)


def _mono_key(x):
    b = jax.lax.bitcast_convert_type(x, jnp.int32)
    return jnp.where(b >= 0, b, b ^ jnp.int32(0x7FFFFFFF))


def _kv_kernel(x_ref, cos_ref, sin_ref, w_ref, wik_ref, k_ref, v_ref, ki_ref):
    x = x_ref[...]
    y = jnp.dot(x.astype(jnp.bfloat16), w_ref[...].astype(jnp.bfloat16),
                preferred_element_type=jnp.float32)
    c = cos_ref[...]
    s = sin_ref[...]
    for h in range(HKV):
        k_ref[:, h * DH:(h + 1) * DH] = _rope_rms_head(
            y[:, h * DH:(h + 1) * DH], c, s)
    v_ref[...] = y[:, HKV * DH:2 * HKV * DH]
    ki_ref[...] = jnp.dot(x, wik_ref[...], preferred_element_type=jnp.float32)


def _make_attn_kernel(width, row0):
    def body(x_ref, cos_ref, sin_ref, wq_ref, wiq_ref, wiw_ref, wo_ref,
             k_ref, v_ref, ki_ref, tri_ref, out_ref):
        blk = pl.program_id(0)
        x = x_ref[...]
        c = cos_ref[...]
        s = sin_ref[...]

        yq = jnp.dot(x.astype(jnp.bfloat16), wq_ref[...].astype(jnp.bfloat16),
                     preferred_element_type=jnp.float32)
        qh = [_rope_rms_head(yq[:, h * DH:(h + 1) * DH], c, s)
              for h in range(H)]

        qi = jnp.dot(x, wiq_ref[...], preferred_element_type=jnp.float32)
        wi = jnp.dot(x, wiw_ref[...], preferred_element_type=jnp.float32)
        ki = ki_ref[...]
        acc = jnp.zeros((BLK, width), jnp.float32)
        for h in range(NIH):
            raw = jax.lax.dot_general(
                qi[:, h * IHD:(h + 1) * IHD], ki,
                (((1,), (1,)), ((), ())), preferred_element_type=jnp.float32)
            acc = acc + jnp.maximum(raw, 0.0) * wi[:, h:h + 1]

        col = jax.lax.broadcasted_iota(jnp.int32, (BLK, width), 1)
        row = jax.lax.broadcasted_iota(jnp.int32, (BLK, width), 0) \
            + (row0 + blk * BLK)
        valid = col <= row
        masked = jnp.where(valid, acc, -jnp.inf)
        key = _mono_key(masked)

        ones_col = jnp.ones((width, 1), jnp.float32)
        NCH = 2
        RC = BLK // NCH
        keys_c = [key[i * RC:(i + 1) * RC] for i in range(NCH)]

        def cnt_ge(kc, v):
            m = jnp.where(kc >= v, 1.0, 0.0)
            return jnp.dot(m, ones_col, preferred_element_type=jnp.float32)

        v64_c = [
            jnp.broadcast_to(
                jnp.where(cnt_ge(keys_c[i],
                                 jnp.zeros((RC, 1), jnp.int32)) >= TOPK,
                          jnp.int32(0), jnp.int32(-2147483648)),
                (RC, 1))
            for i in range(NCH)
        ]
        for bit in range(30, -1, -1):
            for i in range(NCH):
                cand = v64_c[i] | jnp.int32(1 << bit)
                v64_c[i] = jnp.where(cnt_ge(keys_c[i], cand) >= TOPK,
                                     cand, v64_c[i])
        v64 = jnp.concatenate(v64_c, axis=0)

        gt = key > v64
        eq = key == v64
        gt_f = jnp.where(gt, 1.0, 0.0)
        need = TOPK - jnp.dot(gt_f, ones_col,
                              preferred_element_type=jnp.float32)

        eq_b = jnp.where(eq, 1.0, 0.0).astype(jnp.bfloat16)
        pc = jnp.dot(eq_b, tri_ref[...], preferred_element_type=jnp.float32)
        sel = (gt | (eq & (pc < need))) & valid

        sel_st = jnp.concatenate([sel] * G, axis=0)
        oh = [None] * H
        for n in range(HKV):
            kn = k_ref[:, n * DH:(n + 1) * DH]
            vn = v_ref[:, n * DH:(n + 1) * DH]
            ve = jnp.concatenate(
                [vn, jnp.ones((width, 1), jnp.float32)], axis=1)
            q_st = jnp.concatenate(
                [qh[n * G + g] for g in range(G)], axis=0) * SCALE
            sc = jax.lax.dot_general(
                q_st.astype(jnp.bfloat16), kn.astype(jnp.bfloat16),
                (((1,), (1,)), ((), ())),
                preferred_element_type=jnp.float32)
            p = jnp.where(sel_st, jnp.exp(sc), 0.0)
            o_st = jnp.dot(p.astype(jnp.bfloat16), ve.astype(jnp.bfloat16),
                           preferred_element_type=jnp.float32)
            o_st = o_st[:, :DH] / o_st[:, DH:DH + 1]
            for g in range(G):
                oh[n * G + g] = o_st[g * BLK:(g + 1) * BLK, :]

        out_heads = jnp.concatenate(oh, axis=1)
        out_ref[...] = jnp.dot(out_heads.astype(jnp.bfloat16),
                               wo_ref[...].astype(jnp.bfloat16),
                               preferred_element_type=jnp.float32)

    return body


@jax.jit
def kernel(x, cos, sin, Wq, Wk, Wv, Wo, Wiq, Wiw, Wik):
    x2 = x.reshape(T, DM)
    cos2 = cos.reshape(T, DH)
    sin2 = sin.reshape(T, DH)
    wkv = jnp.concatenate([Wk, Wv], axis=1)

    full = lambda shape: pl.BlockSpec(shape, lambda i: (0, 0))
    rows = lambda w: pl.BlockSpec((BLK, w), lambda i: (i, 0))

    k, v, ki = pl.pallas_call(
        _kv_kernel,
        grid=(NBLK,),
        in_specs=[rows(DM), rows(DH), rows(DH), full((DM, 2 * HKV * DH)),
                  full((DM, IHD))],
        out_specs=[rows(HKV * DH), rows(HKV * DH), rows(IHD)],
        out_shape=[
            jax.ShapeDtypeStruct((T, HKV * DH), jnp.float32),
            jax.ShapeDtypeStruct((T, HKV * DH), jnp.float32),
            jax.ShapeDtypeStruct((T, IHD), jnp.float32),
        ],
        compiler_params=pltpu.CompilerParams(
            dimension_semantics=("arbitrary",)),
    )(x2, cos2, sin2, wkv, Wik)

    tri = (jnp.arange(T)[:, None] < jnp.arange(T)[None, :]).astype(jnp.bfloat16)

    outs = []
    for p in range(NBLK // 2):
        row0 = p * 2 * BLK
        width = (p + 1) * 2 * BLK
        o = pl.pallas_call(
            _make_attn_kernel(width, row0),
            grid=(2,),
            in_specs=[
                rows(DM), rows(DH), rows(DH),
                full((DM, H * DH)), full((DM, NIH * IHD)), full((DM, NIH)),
                full((H * DH, DM)),
                full((width, HKV * DH)), full((width, HKV * DH)),
                full((width, IHD)), full((width, width)),
            ],
            out_specs=rows(DM),
            out_shape=jax.ShapeDtypeStruct((2 * BLK, DM), jnp.float32),
            compiler_params=pltpu.CompilerParams(
                dimension_semantics=("arbitrary",)),
        )(x2[row0:row0 + 2 * BLK], cos2[row0:row0 + 2 * BLK],
          sin2[row0:row0 + 2 * BLK], Wq, Wiq, Wiw, Wo,
          k[:width], v[:width], ki[:width], tri[:width, :width])
        outs.append(o)

    return jnp.concatenate(outs, axis=0).reshape(B, T, DM)


# speedup vs baseline: 12.4499x; 1.0178x over previous
import functools

import jax
import jax.numpy as jnp
from jax.experimental import pallas as pl
from jax.experimental.pallas import tpu as pltpu

B, T, DM = 1, 2048, 1024
H, HKV, DH = 16, 4, 64
TOPK = 64
NIH, IHD = 4, 32
EPS = 1.1920929e-07
SCALE = DH ** -0.5
G = H // HKV

BLK = 256
NBLK = T // BLK
NEG = -1e30


def _rope_rms_head(xh, c, s):
    d = DH // 2
    rot = jnp.concatenate([-xh[:, d:], xh[:, :d]], axis=1)
    r = xh * c + rot * s
    return r * jax.lax.rsqrt(jnp.mean(r * r, axis=-1, keepdims=True) + EPS)


def _mono_key(x):
    b = jax.lax.bitcast_convert_type(x, jnp.int32)
    return jnp.where(b >= 0, b, b ^ jnp.int32(0x7FFFFFFF))


def _kv_kernel(x_ref, cos_ref, sin_ref, w_ref, wik_ref, k_ref, v_ref, ki_ref):
    x = x_ref[...]
    y = jnp.dot(x.astype(jnp.bfloat16), w_ref[...].astype(jnp.bfloat16),
                preferred_element_type=jnp.float32)
    c = cos_ref[...]
    s = sin_ref[...]
    for h in range(HKV):
        k_ref[:, h * DH:(h + 1) * DH] = _rope_rms_head(
            y[:, h * DH:(h + 1) * DH], c, s)
    v_ref[...] = y[:, HKV * DH:2 * HKV * DH]
    ki_ref[...] = jnp.dot(x, wik_ref[...], preferred_element_type=jnp.float32)


def _make_attn_kernel(width, row0):
    def body(x_ref, cos_ref, sin_ref, wq_ref, wiq_ref, wiw_ref, wo_ref,
             k_ref, v_ref, ki_ref, tri_ref, out_ref):
        blk = pl.program_id(0)
        x = x_ref[...]
        c = cos_ref[...]
        s = sin_ref[...]

        yq = jnp.dot(x.astype(jnp.bfloat16), wq_ref[...].astype(jnp.bfloat16),
                     preferred_element_type=jnp.float32)
        qh = [_rope_rms_head(yq[:, h * DH:(h + 1) * DH], c, s)
              for h in range(H)]

        qi = jnp.dot(x, wiq_ref[...], preferred_element_type=jnp.float32)
        wi = jnp.dot(x, wiw_ref[...], preferred_element_type=jnp.float32)
        ki = ki_ref[...]
        acc = jnp.zeros((BLK, width), jnp.float32)
        for h in range(NIH):
            raw = jax.lax.dot_general(
                qi[:, h * IHD:(h + 1) * IHD], ki,
                (((1,), (1,)), ((), ())), preferred_element_type=jnp.float32)
            acc = acc + jnp.maximum(raw, 0.0) * wi[:, h:h + 1]

        col = jax.lax.broadcasted_iota(jnp.int32, (BLK, width), 1)
        row = jax.lax.broadcasted_iota(jnp.int32, (BLK, width), 0) \
            + (row0 + blk * BLK)
        valid = col <= row
        masked = jnp.where(valid, acc, -jnp.inf)
        key = _mono_key(masked)

        ones_col = jnp.ones((width, 1), jnp.float32)
        NCH = 4
        RC = BLK // NCH
        keys_c = [key[i * RC:(i + 1) * RC] for i in range(NCH)]

        def cnt_ge(kc, v):
            m = jnp.where(kc >= v, 1.0, 0.0)
            return jnp.dot(m, ones_col, preferred_element_type=jnp.float32)

        v64_c = [
            jnp.broadcast_to(
                jnp.where(cnt_ge(keys_c[i],
                                 jnp.zeros((RC, 1), jnp.int32)) >= TOPK,
                          jnp.int32(0), jnp.int32(-2147483648)),
                (RC, 1))
            for i in range(NCH)
        ]
        for bit in range(30, -1, -1):
            for i in range(NCH):
                cand = v64_c[i] | jnp.int32(1 << bit)
                v64_c[i] = jnp.where(cnt_ge(keys_c[i], cand) >= TOPK,
                                     cand, v64_c[i])
        v64 = jnp.concatenate(v64_c, axis=0)

        gt = key > v64
        eq = key == v64
        gt_f = jnp.where(gt, 1.0, 0.0)
        need = TOPK - jnp.dot(gt_f, ones_col,
                              preferred_element_type=jnp.float32)

        eq_b = jnp.where(eq, 1.0, 0.0).astype(jnp.bfloat16)
        pc = jnp.dot(eq_b, tri_ref[...], preferred_element_type=jnp.float32)
        sel = (gt | (eq & (pc < need))) & valid

        sel_st = jnp.concatenate([sel] * G, axis=0)
        oh = [None] * H
        for n in range(HKV):
            kn = k_ref[:, n * DH:(n + 1) * DH]
            vn = v_ref[:, n * DH:(n + 1) * DH]
            ve = jnp.concatenate(
                [vn, jnp.ones((width, 1), jnp.float32)], axis=1)
            q_st = jnp.concatenate(
                [qh[n * G + g] for g in range(G)], axis=0) * SCALE
            sc = jax.lax.dot_general(
                q_st.astype(jnp.bfloat16), kn.astype(jnp.bfloat16),
                (((1,), (1,)), ((), ())),
                preferred_element_type=jnp.float32)
            p = jnp.where(sel_st, jnp.exp(sc), 0.0)
            o_st = jnp.dot(p.astype(jnp.bfloat16), ve.astype(jnp.bfloat16),
                           preferred_element_type=jnp.float32)
            o_st = o_st[:, :DH] / o_st[:, DH:DH + 1]
            for g in range(G):
                oh[n * G + g] = o_st[g * BLK:(g + 1) * BLK, :]

        out_heads = jnp.concatenate(oh, axis=1)
        out_ref[...] = jnp.dot(out_heads.astype(jnp.bfloat16),
                               wo_ref[...].astype(jnp.bfloat16),
                               preferred_element_type=jnp.float32)

    return body


@jax.jit
def kernel(x, cos, sin, Wq, Wk, Wv, Wo, Wiq, Wiw, Wik):
    x2 = x.reshape(T, DM)
    cos2 = cos.reshape(T, DH)
    sin2 = sin.reshape(T, DH)
    wkv = jnp.concatenate([Wk, Wv], axis=1)

    full = lambda shape: pl.BlockSpec(shape, lambda i: (0, 0))
    rows = lambda w: pl.BlockSpec((BLK, w), lambda i: (i, 0))

    k, v, ki = pl.pallas_call(
        _kv_kernel,
        grid=(NBLK,),
        in_specs=[rows(DM), rows(DH), rows(DH), full((DM, 2 * HKV * DH)),
                  full((DM, IHD))],
        out_specs=[rows(HKV * DH), rows(HKV * DH), rows(IHD)],
        out_shape=[
            jax.ShapeDtypeStruct((T, HKV * DH), jnp.float32),
            jax.ShapeDtypeStruct((T, HKV * DH), jnp.float32),
            jax.ShapeDtypeStruct((T, IHD), jnp.float32),
        ],
        compiler_params=pltpu.CompilerParams(
            dimension_semantics=("arbitrary",)),
    )(x2, cos2, sin2, wkv, Wik)

    tri = (jnp.arange(T)[:, None] < jnp.arange(T)[None, :]).astype(jnp.bfloat16)

    outs = []
    for p in range(NBLK // 2):
        row0 = p * 2 * BLK
        width = (p + 1) * 2 * BLK
        o = pl.pallas_call(
            _make_attn_kernel(width, row0),
            grid=(2,),
            in_specs=[
                rows(DM), rows(DH), rows(DH),
                full((DM, H * DH)), full((DM, NIH * IHD)), full((DM, NIH)),
                full((H * DH, DM)),
                full((width, HKV * DH)), full((width, HKV * DH)),
                full((width, IHD)), full((width, width)),
            ],
            out_specs=rows(DM),
            out_shape=jax.ShapeDtypeStruct((2 * BLK, DM), jnp.float32),
            compiler_params=pltpu.CompilerParams(
                dimension_semantics=("arbitrary",)),
        )(x2[row0:row0 + 2 * BLK], cos2[row0:row0 + 2 * BLK],
          sin2[row0:row0 + 2 * BLK], Wq, Wiq, Wiw, Wo,
          k[:width], v[:width], ki[:width], tri[:width, :width])
        outs.append(o)

    return jnp.concatenate(outs, axis=0).reshape(B, T, DM)


# speedup vs baseline: 15.7721x; 1.2668x over previous
import functools

import jax
import jax.numpy as jnp
from jax.experimental import pallas as pl
from jax.experimental.pallas import tpu as pltpu

B, T, DM = 1, 2048, 1024
H, HKV, DH = 16, 4, 64
TOPK = 64
NIH, IHD = 4, 32
EPS = 1.1920929e-07
SCALE = DH ** -0.5
G = H // HKV

BLK = 512
NBLK = T // BLK
NEG = -1e30


def _rope_rms_head(xh, c, s):
    d = DH // 2
    rot = jnp.concatenate([-xh[:, d:], xh[:, :d]], axis=1)
    r = xh * c + rot * s
    return r * jax.lax.rsqrt(jnp.mean(r * r, axis=-1, keepdims=True) + EPS)


def _mono_key(x):
    b = jax.lax.bitcast_convert_type(x, jnp.int32)
    return jnp.where(b >= 0, b, b ^ jnp.int32(0x7FFFFFFF))


def _kv_kernel(x_ref, cos_ref, sin_ref, w_ref, wik_ref, k_ref, v_ref, ki_ref):
    x = x_ref[...]
    y = jnp.dot(x.astype(jnp.bfloat16), w_ref[...].astype(jnp.bfloat16),
                preferred_element_type=jnp.float32)
    c = cos_ref[...]
    s = sin_ref[...]
    for h in range(HKV):
        k_ref[:, h * DH:(h + 1) * DH] = _rope_rms_head(
            y[:, h * DH:(h + 1) * DH], c, s)
    v_ref[...] = y[:, HKV * DH:2 * HKV * DH]
    ki_ref[...] = jnp.dot(x, wik_ref[...], preferred_element_type=jnp.float32)


def _make_attn_kernel(width, row0):
    def body(x_ref, cos_ref, sin_ref, wq_ref, wiq_ref, wiw_ref, wo_ref,
             k_ref, v_ref, ki_ref, tri_ref, out_ref):
        blk = pl.program_id(0)
        x = x_ref[...]
        c = cos_ref[...]
        s = sin_ref[...]

        yq = jnp.dot(x.astype(jnp.bfloat16), wq_ref[...].astype(jnp.bfloat16),
                     preferred_element_type=jnp.float32)
        qh = [_rope_rms_head(yq[:, h * DH:(h + 1) * DH], c, s)
              for h in range(H)]

        qi = jnp.dot(x, wiq_ref[...], preferred_element_type=jnp.float32)
        wi = jnp.dot(x, wiw_ref[...], preferred_element_type=jnp.float32)
        ki = ki_ref[...]
        acc = jnp.zeros((BLK, width), jnp.float32)
        for h in range(NIH):
            raw = jax.lax.dot_general(
                qi[:, h * IHD:(h + 1) * IHD], ki,
                (((1,), (1,)), ((), ())), preferred_element_type=jnp.float32)
            acc = acc + jnp.maximum(raw, 0.0) * wi[:, h:h + 1]

        col = jax.lax.broadcasted_iota(jnp.int32, (BLK, width), 1)
        row = jax.lax.broadcasted_iota(jnp.int32, (BLK, width), 0) \
            + (row0 + blk * BLK)
        valid = col <= row
        masked = jnp.where(valid, acc, -jnp.inf)
        key = _mono_key(masked)

        ones_col = jnp.ones((width, 1), jnp.float32)
        NCH = 4
        RC = BLK // NCH
        keys_c = [key[i * RC:(i + 1) * RC] for i in range(NCH)]

        def cnt_ge(kc, v):
            m = jnp.where(kc >= v, 1.0, 0.0)
            return jnp.dot(m, ones_col, preferred_element_type=jnp.float32)

        v64_c = [
            jnp.broadcast_to(
                jnp.where(cnt_ge(keys_c[i],
                                 jnp.zeros((RC, 1), jnp.int32)) >= TOPK,
                          jnp.int32(0), jnp.int32(-2147483648)),
                (RC, 1))
            for i in range(NCH)
        ]
        for bit in range(30, -1, -1):
            for i in range(NCH):
                cand = v64_c[i] | jnp.int32(1 << bit)
                v64_c[i] = jnp.where(cnt_ge(keys_c[i], cand) >= TOPK,
                                     cand, v64_c[i])
        v64 = jnp.concatenate(v64_c, axis=0)

        gt = key > v64
        eq = key == v64
        gt_f = jnp.where(gt, 1.0, 0.0)
        need = TOPK - jnp.dot(gt_f, ones_col,
                              preferred_element_type=jnp.float32)

        eq_b = jnp.where(eq, 1.0, 0.0).astype(jnp.bfloat16)
        pc = jnp.dot(eq_b, tri_ref[...], preferred_element_type=jnp.float32)
        sel = (gt | (eq & (pc < need))) & valid

        sel_st = jnp.concatenate([sel] * G, axis=0)
        oh = [None] * H
        for n in range(HKV):
            kn = k_ref[:, n * DH:(n + 1) * DH]
            vn = v_ref[:, n * DH:(n + 1) * DH]
            ve = jnp.concatenate(
                [vn, jnp.ones((width, 1), jnp.float32)], axis=1)
            q_st = jnp.concatenate(
                [qh[n * G + g] for g in range(G)], axis=0) * SCALE
            sc = jax.lax.dot_general(
                q_st.astype(jnp.bfloat16), kn.astype(jnp.bfloat16),
                (((1,), (1,)), ((), ())),
                preferred_element_type=jnp.float32)
            p = jnp.where(sel_st, jnp.exp(sc), 0.0)
            o_st = jnp.dot(p.astype(jnp.bfloat16), ve.astype(jnp.bfloat16),
                           preferred_element_type=jnp.float32)
            o_st = o_st[:, :DH] / o_st[:, DH:DH + 1]
            for g in range(G):
                oh[n * G + g] = o_st[g * BLK:(g + 1) * BLK, :]

        out_heads = jnp.concatenate(oh, axis=1)
        out_ref[...] = jnp.dot(out_heads.astype(jnp.bfloat16),
                               wo_ref[...].astype(jnp.bfloat16),
                               preferred_element_type=jnp.float32)

    return body


@jax.jit
def kernel(x, cos, sin, Wq, Wk, Wv, Wo, Wiq, Wiw, Wik):
    x2 = x.reshape(T, DM)
    cos2 = cos.reshape(T, DH)
    sin2 = sin.reshape(T, DH)
    wkv = jnp.concatenate([Wk, Wv], axis=1)

    full = lambda shape: pl.BlockSpec(shape, lambda i: (0, 0))
    rows = lambda w: pl.BlockSpec((BLK, w), lambda i: (i, 0))

    k, v, ki = pl.pallas_call(
        _kv_kernel,
        grid=(NBLK,),
        in_specs=[rows(DM), rows(DH), rows(DH), full((DM, 2 * HKV * DH)),
                  full((DM, IHD))],
        out_specs=[rows(HKV * DH), rows(HKV * DH), rows(IHD)],
        out_shape=[
            jax.ShapeDtypeStruct((T, HKV * DH), jnp.float32),
            jax.ShapeDtypeStruct((T, HKV * DH), jnp.float32),
            jax.ShapeDtypeStruct((T, IHD), jnp.float32),
        ],
        compiler_params=pltpu.CompilerParams(
            dimension_semantics=("arbitrary",)),
    )(x2, cos2, sin2, wkv, Wik)

    tri = (jnp.arange(T)[:, None] < jnp.arange(T)[None, :]).astype(jnp.bfloat16)

    outs = []
    for p in range(NBLK):
        row0 = p * BLK
        width = (p + 1) * BLK
        o = pl.pallas_call(
            _make_attn_kernel(width, row0),
            grid=(1,),
            in_specs=[
                rows(DM), rows(DH), rows(DH),
                full((DM, H * DH)), full((DM, NIH * IHD)), full((DM, NIH)),
                full((H * DH, DM)),
                full((width, HKV * DH)), full((width, HKV * DH)),
                full((width, IHD)), full((width, width)),
            ],
            out_specs=rows(DM),
            out_shape=jax.ShapeDtypeStruct((BLK, DM), jnp.float32),
            compiler_params=pltpu.CompilerParams(
                dimension_semantics=("arbitrary",)),
        )(x2[row0:row0 + BLK], cos2[row0:row0 + BLK],
          sin2[row0:row0 + BLK], Wq, Wiq, Wiw, Wo,
          k[:width], v[:width], ki[:width], tri[:width, :width])
        outs.append(o)

    return jnp.concatenate(outs, axis=0).reshape(B, T, DM)


# speedup vs baseline: 16.9395x; 1.0740x over previous
import functools

import jax
import jax.numpy as jnp
from jax.experimental import pallas as pl
from jax.experimental.pallas import tpu as pltpu

B, T, DM = 1, 2048, 1024
H, HKV, DH = 16, 4, 64
TOPK = 64
NIH, IHD = 4, 32
EPS = 1.1920929e-07
SCALE = DH ** -0.5
G = H // HKV

BLK = 1024
NBLK = T // BLK
NEG = -1e30


def _rope_rms_head(xh, c, s):
    d = DH // 2
    rot = jnp.concatenate([-xh[:, d:], xh[:, :d]], axis=1)
    r = xh * c + rot * s
    return r * jax.lax.rsqrt(jnp.mean(r * r, axis=-1, keepdims=True) + EPS)


def _mono_key(x):
    b = jax.lax.bitcast_convert_type(x, jnp.int32)
    return jnp.where(b >= 0, b, b ^ jnp.int32(0x7FFFFFFF))


def _kv_kernel(x_ref, cos_ref, sin_ref, w_ref, wik_ref, k_ref, v_ref, ki_ref):
    x = x_ref[...]
    y = jnp.dot(x.astype(jnp.bfloat16), w_ref[...].astype(jnp.bfloat16),
                preferred_element_type=jnp.float32)
    c = cos_ref[...]
    s = sin_ref[...]
    for h in range(HKV):
        k_ref[:, h * DH:(h + 1) * DH] = _rope_rms_head(
            y[:, h * DH:(h + 1) * DH], c, s)
    v_ref[...] = y[:, HKV * DH:2 * HKV * DH]
    ki_ref[...] = jnp.dot(x, wik_ref[...], preferred_element_type=jnp.float32)


def _make_attn_kernel(width, row0):
    def body(x_ref, cos_ref, sin_ref, wq_ref, wiq_ref, wiw_ref, wo_ref,
             k_ref, v_ref, ki_ref, tri_ref, out_ref):
        blk = pl.program_id(0)
        x = x_ref[...]
        c = cos_ref[...]
        s = sin_ref[...]

        yq = jnp.dot(x.astype(jnp.bfloat16), wq_ref[...].astype(jnp.bfloat16),
                     preferred_element_type=jnp.float32)
        qh = [_rope_rms_head(yq[:, h * DH:(h + 1) * DH], c, s)
              for h in range(H)]

        qi = jnp.dot(x, wiq_ref[...], preferred_element_type=jnp.float32)
        wi = jnp.dot(x, wiw_ref[...], preferred_element_type=jnp.float32)
        ki = ki_ref[...]
        acc = jnp.zeros((BLK, width), jnp.float32)
        for h in range(NIH):
            raw = jax.lax.dot_general(
                qi[:, h * IHD:(h + 1) * IHD], ki,
                (((1,), (1,)), ((), ())), preferred_element_type=jnp.float32)
            acc = acc + jnp.maximum(raw, 0.0) * wi[:, h:h + 1]

        col = jax.lax.broadcasted_iota(jnp.int32, (BLK, width), 1)
        row = jax.lax.broadcasted_iota(jnp.int32, (BLK, width), 0) \
            + (row0 + blk * BLK)
        valid = col <= row
        masked = jnp.where(valid, acc, -jnp.inf)
        key = _mono_key(masked)

        ones_col = jnp.ones((width, 1), jnp.float32)
        NCH = 4
        RC = BLK // NCH
        keys_c = [key[i * RC:(i + 1) * RC] for i in range(NCH)]

        def cnt_ge(kc, v):
            m = jnp.where(kc >= v, 1.0, 0.0)
            return jnp.dot(m, ones_col, preferred_element_type=jnp.float32)

        v64_c = [
            jnp.broadcast_to(
                jnp.where(cnt_ge(keys_c[i],
                                 jnp.zeros((RC, 1), jnp.int32)) >= TOPK,
                          jnp.int32(0), jnp.int32(-2147483648)),
                (RC, 1))
            for i in range(NCH)
        ]
        for bit in range(30, -1, -1):
            for i in range(NCH):
                cand = v64_c[i] | jnp.int32(1 << bit)
                v64_c[i] = jnp.where(cnt_ge(keys_c[i], cand) >= TOPK,
                                     cand, v64_c[i])
        v64 = jnp.concatenate(v64_c, axis=0)

        gt = key > v64
        eq = key == v64
        gt_f = jnp.where(gt, 1.0, 0.0)
        need = TOPK - jnp.dot(gt_f, ones_col,
                              preferred_element_type=jnp.float32)

        eq_b = jnp.where(eq, 1.0, 0.0).astype(jnp.bfloat16)
        pc = jnp.dot(eq_b, tri_ref[...], preferred_element_type=jnp.float32)
        sel = (gt | (eq & (pc < need))) & valid

        oh = [None] * H
        for h in range(H):
            n = h // G
            kn = k_ref[:, n * DH:(n + 1) * DH]
            vn = v_ref[:, n * DH:(n + 1) * DH]
            ve = jnp.concatenate(
                [vn, jnp.ones((width, 1), jnp.float32)], axis=1)
            sc = jax.lax.dot_general(
                (qh[h] * SCALE).astype(jnp.bfloat16), kn.astype(jnp.bfloat16),
                (((1,), (1,)), ((), ())),
                preferred_element_type=jnp.float32)
            p = jnp.where(sel, jnp.exp(sc), 0.0)
            o = jnp.dot(p.astype(jnp.bfloat16), ve.astype(jnp.bfloat16),
                        preferred_element_type=jnp.float32)
            oh[h] = o[:, :DH] / o[:, DH:DH + 1]

        out_heads = jnp.concatenate(oh, axis=1)
        out_ref[...] = jnp.dot(out_heads.astype(jnp.bfloat16),
                               wo_ref[...].astype(jnp.bfloat16),
                               preferred_element_type=jnp.float32)

    return body


@jax.jit
def kernel(x, cos, sin, Wq, Wk, Wv, Wo, Wiq, Wiw, Wik):
    x2 = x.reshape(T, DM)
    cos2 = cos.reshape(T, DH)
    sin2 = sin.reshape(T, DH)
    wkv = jnp.concatenate([Wk, Wv], axis=1)

    full = lambda shape: pl.BlockSpec(shape, lambda i: (0, 0))
    rows = lambda w: pl.BlockSpec((BLK, w), lambda i: (i, 0))

    k, v, ki = pl.pallas_call(
        _kv_kernel,
        grid=(NBLK,),
        in_specs=[rows(DM), rows(DH), rows(DH), full((DM, 2 * HKV * DH)),
                  full((DM, IHD))],
        out_specs=[rows(HKV * DH), rows(HKV * DH), rows(IHD)],
        out_shape=[
            jax.ShapeDtypeStruct((T, HKV * DH), jnp.float32),
            jax.ShapeDtypeStruct((T, HKV * DH), jnp.float32),
            jax.ShapeDtypeStruct((T, IHD), jnp.float32),
        ],
        compiler_params=pltpu.CompilerParams(
            dimension_semantics=("arbitrary",)),
    )(x2, cos2, sin2, wkv, Wik)

    tri = (jnp.arange(T)[:, None] < jnp.arange(T)[None, :]).astype(jnp.bfloat16)

    outs = []
    for p in range(NBLK):
        row0 = p * BLK
        width = (p + 1) * BLK
        o = pl.pallas_call(
            _make_attn_kernel(width, row0),
            grid=(1,),
            in_specs=[
                rows(DM), rows(DH), rows(DH),
                full((DM, H * DH)), full((DM, NIH * IHD)), full((DM, NIH)),
                full((H * DH, DM)),
                full((width, HKV * DH)), full((width, HKV * DH)),
                full((width, IHD)), full((width, width)),
            ],
            out_specs=rows(DM),
            out_shape=jax.ShapeDtypeStruct((BLK, DM), jnp.float32),
            compiler_params=pltpu.CompilerParams(
                dimension_semantics=("arbitrary",)),
        )(x2[row0:row0 + BLK], cos2[row0:row0 + BLK],
          sin2[row0:row0 + BLK], Wq, Wiq, Wiw, Wo,
          k[:width], v[:width], ki[:width], tri[:width, :width])
        outs.append(o)

    return jnp.concatenate(outs, axis=0).reshape(B, T, DM)
